# baseline (speedup 1.0000x reference)
import functools

import jax
import jax.numpy as jnp
from jax import lax
from jax.experimental import pallas as pl
from jax.experimental.pallas import tpu as pltpu
from jax.experimental.pallas import tpu_sc as plsc

B, N, C, E = 2, 2048, 128, 4
K = N // 2
L = 16
NC, NS = 2, 16
NW = NC * NS
RPW = (B * K) // NW
RCHUNK = 4
NCHUNK = RPW // RCHUNK


def _topk_body(x_ref, w1_ref, b1_ref, w2_ref, b2_ref, w3t_ref, idx_ref):
    xb = x_ref[0]
    h = jnp.maximum(jnp.dot(xb, w1_ref[...], preferred_element_type=jnp.float32) + b1_ref[0], 0.0)
    h = jnp.maximum(jnp.dot(h, w2_ref[...], preferred_element_type=jnp.float32) + b2_ref[0], 0.0)
    s_bcast = jnp.dot(h, w3t_ref[...], preferred_element_type=jnp.float32)
    s_col = s_bcast[:, :1]
    s_row = jnp.transpose(s_bcast)[:1, :]

    j_iota = lax.broadcasted_iota(jnp.int32, (1, N), 1).astype(jnp.float32)
    cb = 256
    rank_chunks = []
    for t in range(N // cb):
        sc = s_col[t * cb:(t + 1) * cb, :]
        i_iota = lax.broadcasted_iota(jnp.int32, (cb, 1), 0).astype(jnp.float32) + float(t * cb)
        gt = jnp.sum((s_row > sc).astype(jnp.float32), axis=1, keepdims=True)
        eq = jnp.sum(((s_row == sc) & (j_iota < i_iota)).astype(jnp.float32),
                     axis=1, keepdims=True)
        rank_chunks.append(gt + eq)
    ranks = jnp.concatenate(rank_chunks, axis=0)

    p_iota = lax.broadcasted_iota(jnp.int32, (1, K), 1).astype(jnp.float32)
    idx_vals = jnp.zeros((1, K), jnp.float32)
    for t in range(N // cb):
        rc = ranks[t * cb:(t + 1) * cb, :]
        i_iota = lax.broadcasted_iota(jnp.int32, (cb, 1), 0).astype(jnp.float32) + float(t * cb)
        match = (rc == p_iota).astype(jnp.float32)
        idx_vals = idx_vals + jnp.sum(match * i_iota, axis=0, keepdims=True)
    idx_ref[0] = idx_vals.astype(jnp.int32)


def _topk_idx(x, W1, b1, W2, b2, W3):
    w3t = jnp.tile(W3, (1, 128))
    return pl.pallas_call(
        _topk_body,
        grid=(B,),
        in_specs=[
            pl.BlockSpec((1, N, C), lambda b: (b, 0, 0)),
            pl.BlockSpec((C, 64), lambda b: (0, 0)),
            pl.BlockSpec((1, 64), lambda b: (0, 0)),
            pl.BlockSpec((64, 16), lambda b: (0, 0)),
            pl.BlockSpec((1, 16), lambda b: (0, 0)),
            pl.BlockSpec((16, 128), lambda b: (0, 0)),
        ],
        out_specs=pl.BlockSpec((1, 1, K), lambda b: (b, 0, 0)),
        out_shape=jax.ShapeDtypeStruct((B, 1, K), jnp.int32),
    )(x, W1, b1.reshape(1, 64), W2, b2.reshape(1, 16), w3t).reshape(B, K)


def _sc_body(colidx_hbm, ecolidx_hbm, rowids2d_hbm, rowids_hbm,
             x_hbm, adj_hbm, edge_hbm,
             feat_out, adj_out, edge_out,
             colidx_v, ecolidx_v, rowids_v, rowflat_v, feat_v,
             adjrows_v, edgerows_v, oadj_v, oedge_v,
             sem_a, sem_e):
    wid = lax.axis_index("s") * NC + lax.axis_index("c")
    base = wid * RPW
    b = base // K

    pltpu.sync_copy(colidx_hbm.at[pl.ds(b * K, K)], colidx_v)
    pltpu.sync_copy(ecolidx_hbm.at[pl.ds(b * K * E, K * E)], ecolidx_v)
    pltpu.sync_copy(rowids2d_hbm.at[wid], rowids_v)
    pltpu.sync_copy(rowids_hbm.at[pl.ds(base, RPW)], rowflat_v)

    pltpu.async_copy(x_hbm.at[rowflat_v], feat_v, sem_a).wait()
    pltpu.sync_copy(feat_v, feat_out.at[pl.ds(base, RPW)])

    def chunk_body(g, carry):
        rid = rowids_v.at[g]
        cp_a = pltpu.async_copy(adj_hbm.at[rid], adjrows_v, sem_a)
        cp_e = pltpu.async_copy(edge_hbm.at[rid], edgerows_v, sem_e)
        cp_a.wait()

        def col_adj(t, c2):
            ci = colidx_v[pl.ds(t * L, L)]
            for r in range(RCHUNK):
                rv = jnp.full((L,), r, jnp.int32)
                oadj_v[r, pl.ds(t * L, L)] = plsc.load_gather(adjrows_v, [rv, ci])
            return c2
        lax.fori_loop(0, K // L, col_adj, 0, unroll=2)
        cp_e.wait()

        def col_edge(t, c2):
            ci = ecolidx_v[pl.ds(t * L, L)]
            for r in range(RCHUNK):
                rv = jnp.full((L,), r, jnp.int32)
                oedge_v[r, pl.ds(t * L, L)] = plsc.load_gather(edgerows_v, [rv, ci])
            return c2
        lax.fori_loop(0, (K * E) // L, col_edge, 0, unroll=2)

        pltpu.sync_copy(oadj_v, adj_out.at[pl.ds(base + g * RCHUNK, RCHUNK)])
        pltpu.sync_copy(oedge_v, edge_out.at[pl.ds(base + g * RCHUNK, RCHUNK)])
        return carry

    lax.fori_loop(0, NCHUNK, chunk_body, 0)


@functools.partial(jax.jit, static_argnames=())
def _sc_pool(colidx, ecolidx, rowids2d, rowids, x2d, adj2d, edge2d):
    mesh = plsc.VectorSubcoreMesh(core_axis_name="c", subcore_axis_name="s",
                                  num_cores=NC, num_subcores=NS)
    f = pl.kernel(
        _sc_body,
        out_type=[
            jax.ShapeDtypeStruct((B * K, C), jnp.float32),
            jax.ShapeDtypeStruct((B * K, K), jnp.float32),
            jax.ShapeDtypeStruct((B * K, K * E), jnp.float32),
        ],
        mesh=mesh,
        compiler_params=pltpu.CompilerParams(needs_layout_passes=False),
        scratch_types=[
            pltpu.VMEM((K,), jnp.int32),
            pltpu.VMEM((K * E,), jnp.int32),
            pltpu.VMEM((NCHUNK, RCHUNK), jnp.int32),
            pltpu.VMEM((RPW,), jnp.int32),
            pltpu.VMEM((RPW, C), jnp.float32),
            pltpu.VMEM((RCHUNK, N), jnp.float32),
            pltpu.VMEM((RCHUNK, N * E), jnp.float32),
            pltpu.VMEM((RCHUNK, K), jnp.float32),
            pltpu.VMEM((RCHUNK, K * E), jnp.float32),
            pltpu.SemaphoreType.DMA,
            pltpu.SemaphoreType.DMA,
        ],
    )
    return f(colidx, ecolidx, rowids2d, rowids, x2d, adj2d, edge2d)


def kernel(x, adjacency, edge_features, superpoint_centroids,
           W1, b1, W2, b2, W3, b3):
    del superpoint_centroids, b3
    idx = _topk_idx(x, W1, b1, W2, b2, W3)

    rowids = (idx + jnp.arange(B, dtype=jnp.int32)[:, None] * N).reshape(-1)
    rowids2d = rowids.reshape(NW, NCHUNK, RCHUNK)
    colidx = idx.reshape(-1)
    ecolidx = (idx[:, :, None] * E
               + jnp.arange(E, dtype=jnp.int32)[None, None, :]).reshape(-1)

    feat, adj_p, edge_p = _sc_pool(
        colidx, ecolidx, rowids2d, rowids,
        x.reshape(B * N, C),
        adjacency.reshape(B * N, N),
        edge_features.reshape(B * N, N * E),
    )
    return (feat.reshape(B, K, C),
            adj_p.reshape(B, K, K),
            edge_p.reshape(B, K, K, E))
